# baseline (speedup 1.0000x reference)
import functools

import jax
import jax.numpy as jnp
from jax import lax
from jax.experimental import pallas as pl
from jax.experimental.pallas import tpu as pltpu
from jax.experimental.pallas import tpu_sc as plsc

B = 4096
V = 100000
D = 16
L = 50
NC = 2
NS = 16
NW = NC * NS
BPT = B // NW
INV_L = 1.0 / L
KC = 800
NCH = V // KC
RPT = NCH // NW
REM = NCH - RPT * NW
NT = 12


def _tree_sum(vals):
    while len(vals) > 1:
        nxt = [vals[i] + vals[i + 1] for i in range(0, len(vals) - 1, 2)]
        if len(vals) % 2:
            nxt.append(vals[-1])
        vals = nxt
    return vals[0]


def kernel(input_0, input_1, input_2, input_3,
           W_0_1, W_0_2, W_0_3,
           W_1_0, W_1_2, W_1_3,
           W_2_0, W_2_1, W_2_3,
           W_3_0, W_3_1, W_3_2):
    mesh = plsc.VectorSubcoreMesh(core_axis_name="c", subcore_axis_name="s")

    @functools.partial(
        pl.kernel,
        mesh=mesh,
        compiler_params=pltpu.CompilerParams(
            needs_layout_passes=False, use_tc_tiling_on_sc=False),
        out_type=tuple(jax.ShapeDtypeStruct((V, D), jnp.float32)
                       for _ in range(NT)),
        scratch_types=[
            pltpu.VMEM((D * KC,), jnp.float32),
            pltpu.VMEM((D * KC,), jnp.float32),
            pltpu.VMEM((KC, D), jnp.float32),
            pltpu.VMEM((KC, D), jnp.float32),
            pltpu.SemaphoreType.DMA,
            pltpu.SemaphoreType.DMA,
            pltpu.SemaphoreType.DMA,
            pltpu.SemaphoreType.DMA,
        ],
    )
    def k1(*refs):
        wts = refs[:NT]
        outs = refs[NT:2 * NT]
        colA, colB, rowA, rowB, semA, semB, semOA, semOB = refs[2 * NT:]
        wid = lax.axis_index("s") * NC + lax.axis_index("c")
        lanes = lax.iota(jnp.int32, D)
        base_idx = KC * lanes

        def issue_load(t, r, col, sem):
            off = (wid + r * NW) * KC
            wt = wts[t]
            for f in range(D):
                pltpu.make_async_copy(
                    wt.at[pl.ds(f * V + off, KC)],
                    col.at[pl.ds(f * KC, KC)], sem).start()

        def wait_load(col, sem):
            for f in range(D):
                pltpu.make_async_copy(
                    wts[0].at[pl.ds(0, KC)],
                    col.at[pl.ds(f * KC, KC)], sem).wait()

        def transpose(col, row):
            def body(j, carry):
                j8 = j * 8
                for u in range(8):
                    row[j8 + u] = plsc.load_gather(col, [j8 + u + base_idx])
                return carry
            lax.fori_loop(0, KC // 8, body, 0)

        def issue_store(t, r, row, sem):
            c = wid + r * NW
            pltpu.make_async_copy(row, outs[t].at[pl.ds(c * KC, KC)],
                                  sem).start()

        def wait_store(row, sem):
            pltpu.make_async_copy(row, outs[0].at[pl.ds(0, KC)], sem).wait()

        bufs = [(colA, rowA, semA, semOA), (colB, rowB, semB, semOB)]

        def round_body(r, carry):
            issue_load(0, r, colA, semA)
            for t in range(NT):
                col, row, sem, semO = bufs[t % 2]
                ncol, _, nsem, _ = bufs[(t + 1) % 2]
                wait_load(col, sem)
                if t + 1 < NT:
                    issue_load(t + 1, r, ncol, nsem)
                if t < 2:
                    pl.when(r > 0)(functools.partial(wait_store, row, semO))
                else:
                    wait_store(row, semO)
                transpose(col, row)
                issue_store(t, r, row, semO)
            return carry

        lax.fori_loop(0, RPT, round_body, 0)

        @pl.when(wid < REM)
        def _():
            round_body(RPT, 0)

        wait_store(rowA, semOA)
        wait_store(rowB, semOB)

    @functools.partial(
        pl.kernel,
        mesh=mesh,
        compiler_params=pltpu.CompilerParams(use_tc_tiling_on_sc=False),
        out_type=jax.ShapeDtypeStruct((B,), jnp.float32),
        scratch_types=[
            pltpu.VMEM((BPT,), jnp.int32),
            pltpu.VMEM((BPT,), jnp.int32),
            pltpu.VMEM((BPT,), jnp.int32),
            pltpu.VMEM((BPT, L), jnp.int32),
        ] + [pltpu.VMEM((BPT, D), jnp.float32) for _ in range(9)]
          + [pltpu.VMEM((L, D), jnp.float32) for _ in range(6)]
          + [
            pltpu.VMEM((BPT,), jnp.float32),
            pltpu.SemaphoreType.DMA,
            pltpu.SemaphoreType.DMA,
            pltpu.SemaphoreType.DMA,
        ],
    )
    def k2(i0, i1, i2, i3,
           w01, w02, w03, w10, w12, w13, w20, w21, w23, w30, w31, w32,
           out,
           idx0_v, idx1_v, idx2_v, idx3_v,
           r01, r02, r03, r10, r12, r13, r20, r21, r23,
           s0a, s1a, s2a, s0b, s1b, s2b,
           out_v,
           sem_sc, sem_a, sem_b):
        wid = lax.axis_index("s") * NC + lax.axis_index("c")
        base = wid * BPT

        pltpu.sync_copy(i0.at[pl.ds(base, BPT)], idx0_v)
        pltpu.sync_copy(i1.at[pl.ds(base, BPT)], idx1_v)
        pltpu.sync_copy(i2.at[pl.ds(base, BPT)], idx2_v)
        pltpu.sync_copy(i3.at[pl.ds(base, BPT)], idx3_v)

        sc_copies = [
            pltpu.make_async_copy(w01.at[idx0_v], r01, sem_sc),
            pltpu.make_async_copy(w02.at[idx0_v], r02, sem_sc),
            pltpu.make_async_copy(w03.at[idx0_v], r03, sem_sc),
            pltpu.make_async_copy(w10.at[idx1_v], r10, sem_sc),
            pltpu.make_async_copy(w12.at[idx1_v], r12, sem_sc),
            pltpu.make_async_copy(w13.at[idx1_v], r13, sem_sc),
            pltpu.make_async_copy(w20.at[idx2_v], r20, sem_sc),
            pltpu.make_async_copy(w21.at[idx2_v], r21, sem_sc),
            pltpu.make_async_copy(w23.at[idx2_v], r23, sem_sc),
        ]
        for c in sc_copies:
            c.start()
        for c in sc_copies:
            c.wait()

        ring0 = (s0a, s1a, s2a)
        ring1 = (s0b, s1b, s2b)

        def issue(b, bufs, sem):
            idxrow = idx3_v.at[b]
            pltpu.make_async_copy(w30.at[idxrow], bufs[0], sem).start()
            pltpu.make_async_copy(w31.at[idxrow], bufs[1], sem).start()
            pltpu.make_async_copy(w32.at[idxrow], bufs[2], sem).start()

        def wait3(bufs, sem):
            for buf in bufs:
                pltpu.make_async_copy(w30.at[idx3_v.at[0]], buf, sem).wait()

        lanes = lax.iota(jnp.int32, D)
        perms = [lanes ^ sh for sh in (8, 4, 2, 1)]

        gdn = lax.GatherDimensionNumbers(
            offset_dims=(), collapsed_slice_dims=(0,), start_index_map=(0,))

        def allsum(v):
            for perm in perms:
                v = v + lax.gather(
                    v, perm[:, None], dimension_numbers=gdn, slice_sizes=(1,),
                    mode=lax.GatherScatterMode.PROMISE_IN_BOUNDS)
            return v

        def compute(b, bufs):
            m0 = _tree_sum([bufs[0][l] for l in range(L)])
            m1 = _tree_sum([bufs[1][l] for l in range(L)])
            m2 = _tree_sum([bufs[2][l] for l in range(L)])
            p = (r01[b] * r10[b] + r02[b] * r20[b] + r12[b] * r21[b]
                 + (r03[b] * m0 + r13[b] * m1 + r23[b] * m2) * INV_L)
            return allsum(p)

        issue(0, ring0, sem_a)

        def body(t, acc):
            b0 = 2 * t
            lane0 = b0 % D
            issue(b0 + 1, ring1, sem_b)
            wait3(ring0, sem_a)
            s0 = compute(b0, ring0)
            issue(jnp.minimum(b0 + 2, BPT - 1), ring0, sem_a)
            wait3(ring1, sem_b)
            s1 = compute(b0 + 1, ring1)
            acc = jnp.where(lanes == lane0, s0, acc)
            acc = jnp.where(lanes == lane0 + 1, s1, acc)

            @pl.when(t % (D // 2) == (D // 2) - 1)
            def _():
                out_v[pl.ds((t // (D // 2)) * D, D)] = acc

            return acc

        lax.fori_loop(0, BPT // 2, body, jnp.zeros((D,), jnp.float32))
        wait3(ring0, sem_a)

        pltpu.sync_copy(out_v, out.at[pl.ds(base, BPT)])

    tabs = k1(*[w.T.reshape(V * D) for w in (
        W_0_1, W_0_2, W_0_3, W_1_0, W_1_2, W_1_3,
        W_2_0, W_2_1, W_2_3, W_3_0, W_3_1, W_3_2)])
    out_flat = k2(input_0.reshape(B), input_1.reshape(B), input_2.reshape(B),
                  input_3, *tabs)
    return out_flat.reshape(B, 1, 1)


# speedup vs baseline: 1.0007x; 1.0007x over previous
import functools

import jax
import jax.numpy as jnp
from jax import lax
from jax.experimental import pallas as pl
from jax.experimental.pallas import tpu as pltpu
from jax.experimental.pallas import tpu_sc as plsc

B = 4096
V = 100000
D = 16
L = 50
NC = 2
NS = 16
NW = NC * NS
BPT = B // NW
INV_L = 1.0 / L
KC = 800
NCH = V // KC
RPT = NCH // NW
REM = NCH - RPT * NW
NT = 12


def _tree_sum(vals):
    while len(vals) > 1:
        nxt = [vals[i] + vals[i + 1] for i in range(0, len(vals) - 1, 2)]
        if len(vals) % 2:
            nxt.append(vals[-1])
        vals = nxt
    return vals[0]


def kernel(input_0, input_1, input_2, input_3,
           W_0_1, W_0_2, W_0_3,
           W_1_0, W_1_2, W_1_3,
           W_2_0, W_2_1, W_2_3,
           W_3_0, W_3_1, W_3_2):
    mesh = plsc.VectorSubcoreMesh(core_axis_name="c", subcore_axis_name="s")

    @functools.partial(
        pl.kernel,
        mesh=mesh,
        compiler_params=pltpu.CompilerParams(
            needs_layout_passes=False, use_tc_tiling_on_sc=False),
        out_type=tuple(jax.ShapeDtypeStruct((V, D), jnp.float32)
                       for _ in range(NT)),
        scratch_types=[
            pltpu.VMEM((D * KC,), jnp.float32),
            pltpu.VMEM((D * KC,), jnp.float32),
            pltpu.VMEM((D * KC,), jnp.float32),
            pltpu.VMEM((KC, D), jnp.float32),
            pltpu.VMEM((KC, D), jnp.float32),
            pltpu.SemaphoreType.DMA,
            pltpu.SemaphoreType.DMA,
            pltpu.SemaphoreType.DMA,
            pltpu.SemaphoreType.DMA,
            pltpu.SemaphoreType.DMA,
        ],
    )
    def k1(*refs):
        wts = refs[:NT]
        outs = refs[NT:2 * NT]
        (colA, colB, colC, rowA, rowB,
         semA, semB, semC, semOA, semOB) = refs[2 * NT:]
        wid = lax.axis_index("s") * NC + lax.axis_index("c")
        lanes = lax.iota(jnp.int32, D)
        base_idx = KC * lanes

        def issue_load(t, r, col, sem):
            off = (wid + r * NW) * KC
            wt = wts[t]
            for f in range(D):
                pltpu.make_async_copy(
                    wt.at[pl.ds(f * V + off, KC)],
                    col.at[pl.ds(f * KC, KC)], sem).start()

        def wait_load(col, sem):
            pltpu.make_async_copy(wts[0].at[pl.ds(0, D * KC)], col, sem).wait()

        def transpose(col, row):
            def body(j, carry):
                j8 = j * 8
                for u in range(8):
                    row[j8 + u] = plsc.load_gather(col, [j8 + u + base_idx])
                return carry
            lax.fori_loop(0, KC // 8, body, 0)

        def issue_store(t, r, row, sem):
            c = wid + r * NW
            pltpu.make_async_copy(row, outs[t].at[pl.ds(c * KC, KC)],
                                  sem).start()

        def wait_store(row, sem):
            pltpu.make_async_copy(row, outs[0].at[pl.ds(0, KC)], sem).wait()

        cols = [(colA, semA), (colB, semB), (colC, semC)]
        rows = [(rowA, semOA), (rowB, semOB)]

        def round_body(r, carry):
            issue_load(0, r, *cols[0])
            issue_load(1, r, *cols[1])
            for t in range(NT):
                col, sem = cols[t % 3]
                row, semO = rows[t % 2]
                wait_load(col, sem)
                if t + 2 < NT:
                    issue_load(t + 2, r, *cols[(t + 2) % 3])
                if t < 2:
                    pl.when(r > 0)(functools.partial(wait_store, row, semO))
                else:
                    wait_store(row, semO)
                transpose(col, row)
                issue_store(t, r, row, semO)
            return carry

        lax.fori_loop(0, RPT, round_body, 0)

        @pl.when(wid < REM)
        def _():
            round_body(RPT, 0)

        wait_store(rowA, semOA)
        wait_store(rowB, semOB)

    @functools.partial(
        pl.kernel,
        mesh=mesh,
        compiler_params=pltpu.CompilerParams(use_tc_tiling_on_sc=False),
        out_type=jax.ShapeDtypeStruct((B,), jnp.float32),
        scratch_types=[
            pltpu.VMEM((BPT,), jnp.int32),
            pltpu.VMEM((BPT,), jnp.int32),
            pltpu.VMEM((BPT,), jnp.int32),
            pltpu.VMEM((BPT, L), jnp.int32),
        ] + [pltpu.VMEM((BPT, D), jnp.float32) for _ in range(9)]
          + [pltpu.VMEM((L, D), jnp.float32) for _ in range(6)]
          + [
            pltpu.VMEM((BPT,), jnp.float32),
            pltpu.SemaphoreType.DMA,
            pltpu.SemaphoreType.DMA,
            pltpu.SemaphoreType.DMA,
        ],
    )
    def k2(i0, i1, i2, i3,
           w01, w02, w03, w10, w12, w13, w20, w21, w23, w30, w31, w32,
           out,
           idx0_v, idx1_v, idx2_v, idx3_v,
           r01, r02, r03, r10, r12, r13, r20, r21, r23,
           s0a, s1a, s2a, s0b, s1b, s2b,
           out_v,
           sem_sc, sem_a, sem_b):
        wid = lax.axis_index("s") * NC + lax.axis_index("c")
        base = wid * BPT

        pltpu.sync_copy(i0.at[pl.ds(base, BPT)], idx0_v)
        pltpu.sync_copy(i1.at[pl.ds(base, BPT)], idx1_v)
        pltpu.sync_copy(i2.at[pl.ds(base, BPT)], idx2_v)
        pltpu.sync_copy(i3.at[pl.ds(base, BPT)], idx3_v)

        sc_copies = [
            pltpu.make_async_copy(w01.at[idx0_v], r01, sem_sc),
            pltpu.make_async_copy(w02.at[idx0_v], r02, sem_sc),
            pltpu.make_async_copy(w03.at[idx0_v], r03, sem_sc),
            pltpu.make_async_copy(w10.at[idx1_v], r10, sem_sc),
            pltpu.make_async_copy(w12.at[idx1_v], r12, sem_sc),
            pltpu.make_async_copy(w13.at[idx1_v], r13, sem_sc),
            pltpu.make_async_copy(w20.at[idx2_v], r20, sem_sc),
            pltpu.make_async_copy(w21.at[idx2_v], r21, sem_sc),
            pltpu.make_async_copy(w23.at[idx2_v], r23, sem_sc),
        ]
        for c in sc_copies:
            c.start()
        for c in sc_copies:
            c.wait()

        ring0 = (s0a, s1a, s2a)
        ring1 = (s0b, s1b, s2b)

        def issue(b, bufs, sem):
            idxrow = idx3_v.at[b]
            pltpu.make_async_copy(w30.at[idxrow], bufs[0], sem).start()
            pltpu.make_async_copy(w31.at[idxrow], bufs[1], sem).start()
            pltpu.make_async_copy(w32.at[idxrow], bufs[2], sem).start()

        def wait3(bufs, sem):
            for buf in bufs:
                pltpu.make_async_copy(w30.at[idx3_v.at[0]], buf, sem).wait()

        lanes = lax.iota(jnp.int32, D)
        perms = [lanes ^ sh for sh in (8, 4, 2, 1)]

        gdn = lax.GatherDimensionNumbers(
            offset_dims=(), collapsed_slice_dims=(0,), start_index_map=(0,))

        def allsum(v):
            for perm in perms:
                v = v + lax.gather(
                    v, perm[:, None], dimension_numbers=gdn, slice_sizes=(1,),
                    mode=lax.GatherScatterMode.PROMISE_IN_BOUNDS)
            return v

        def compute(b, bufs):
            m0 = _tree_sum([bufs[0][l] for l in range(L)])
            m1 = _tree_sum([bufs[1][l] for l in range(L)])
            m2 = _tree_sum([bufs[2][l] for l in range(L)])
            p = (r01[b] * r10[b] + r02[b] * r20[b] + r12[b] * r21[b]
                 + (r03[b] * m0 + r13[b] * m1 + r23[b] * m2) * INV_L)
            return allsum(p)

        issue(0, ring0, sem_a)

        def body(t, acc):
            b0 = 2 * t
            lane0 = b0 % D
            issue(b0 + 1, ring1, sem_b)
            wait3(ring0, sem_a)
            s0 = compute(b0, ring0)
            issue(jnp.minimum(b0 + 2, BPT - 1), ring0, sem_a)
            wait3(ring1, sem_b)
            s1 = compute(b0 + 1, ring1)
            acc = jnp.where(lanes == lane0, s0, acc)
            acc = jnp.where(lanes == lane0 + 1, s1, acc)

            @pl.when(t % (D // 2) == (D // 2) - 1)
            def _():
                out_v[pl.ds((t // (D // 2)) * D, D)] = acc

            return acc

        lax.fori_loop(0, BPT // 2, body, jnp.zeros((D,), jnp.float32))
        wait3(ring0, sem_a)

        pltpu.sync_copy(out_v, out.at[pl.ds(base, BPT)])

    tabs = k1(*[w.T.reshape(V * D) for w in (
        W_0_1, W_0_2, W_0_3, W_1_0, W_1_2, W_1_3,
        W_2_0, W_2_1, W_2_3, W_3_0, W_3_1, W_3_2)])
    out_flat = k2(input_0.reshape(B), input_1.reshape(B), input_2.reshape(B),
                  input_3, *tabs)
    return out_flat.reshape(B, 1, 1)


# speedup vs baseline: 1.2867x; 1.2859x over previous
import functools

import jax
import jax.numpy as jnp
from jax import lax
from jax.experimental import pallas as pl
from jax.experimental.pallas import tpu as pltpu
from jax.experimental.pallas import tpu_sc as plsc

B = 4096
V = 100000
D = 16
L = 50
NC = 2
NS = 16
NW = NC * NS
BPT = B // NW
INV_L = 1.0 / L


def _tree_sum(vals):
    while len(vals) > 1:
        nxt = [vals[i] + vals[i + 1] for i in range(0, len(vals) - 1, 2)]
        if len(vals) % 2:
            nxt.append(vals[-1])
        vals = nxt
    return vals[0]


def kernel(input_0, input_1, input_2, input_3,
           W_0_1, W_0_2, W_0_3,
           W_1_0, W_1_2, W_1_3,
           W_2_0, W_2_1, W_2_3,
           W_3_0, W_3_1, W_3_2):
    mesh = plsc.VectorSubcoreMesh(core_axis_name="c", subcore_axis_name="s")

    @functools.partial(
        pl.kernel,
        mesh=mesh,
        compiler_params=pltpu.CompilerParams(use_tc_tiling_on_sc=False),
        out_type=jax.ShapeDtypeStruct((B,), jnp.float32),
        scratch_types=[
            pltpu.VMEM((BPT,), jnp.int32),
            pltpu.VMEM((BPT,), jnp.int32),
            pltpu.VMEM((BPT,), jnp.int32),
            pltpu.VMEM((BPT, L), jnp.int32),
        ] + [pltpu.VMEM((BPT, D), jnp.float32) for _ in range(9)]
          + [pltpu.VMEM((L, D), jnp.float32) for _ in range(6)]
          + [
            pltpu.VMEM((BPT,), jnp.float32),
            pltpu.SemaphoreType.DMA,
            pltpu.SemaphoreType.DMA,
            pltpu.SemaphoreType.DMA,
        ],
    )
    def k(i0, i1, i2, i3,
          w01, w02, w03, w10, w12, w13, w20, w21, w23, w30, w31, w32,
          out,
          idx0_v, idx1_v, idx2_v, idx3_v,
          r01, r02, r03, r10, r12, r13, r20, r21, r23,
          s0a, s1a, s2a, s0b, s1b, s2b,
          out_v,
          sem_sc, sem_a, sem_b):
        wid = lax.axis_index("s") * NC + lax.axis_index("c")
        base = wid * BPT

        pltpu.sync_copy(i0.at[pl.ds(base, BPT)], idx0_v)
        pltpu.sync_copy(i1.at[pl.ds(base, BPT)], idx1_v)
        pltpu.sync_copy(i2.at[pl.ds(base, BPT)], idx2_v)
        pltpu.sync_copy(i3.at[pl.ds(base, BPT)], idx3_v)

        sc_copies = [
            pltpu.make_async_copy(w01.at[idx0_v], r01, sem_sc),
            pltpu.make_async_copy(w02.at[idx0_v], r02, sem_sc),
            pltpu.make_async_copy(w03.at[idx0_v], r03, sem_sc),
            pltpu.make_async_copy(w10.at[idx1_v], r10, sem_sc),
            pltpu.make_async_copy(w12.at[idx1_v], r12, sem_sc),
            pltpu.make_async_copy(w13.at[idx1_v], r13, sem_sc),
            pltpu.make_async_copy(w20.at[idx2_v], r20, sem_sc),
            pltpu.make_async_copy(w21.at[idx2_v], r21, sem_sc),
            pltpu.make_async_copy(w23.at[idx2_v], r23, sem_sc),
        ]
        for c in sc_copies:
            c.start()
        for c in sc_copies:
            c.wait()

        ring0 = (s0a, s1a, s2a)
        ring1 = (s0b, s1b, s2b)

        def issue(b, bufs, sem):
            idxrow = idx3_v.at[b]
            pltpu.make_async_copy(w30.at[idxrow], bufs[0], sem).start()
            pltpu.make_async_copy(w31.at[idxrow], bufs[1], sem).start()
            pltpu.make_async_copy(w32.at[idxrow], bufs[2], sem).start()

        def wait3(bufs, sem):
            for buf in bufs:
                pltpu.make_async_copy(w30.at[idx3_v.at[0]], buf, sem).wait()

        lanes = lax.iota(jnp.int32, D)
        perms = [lanes ^ sh for sh in (8, 4, 2, 1)]

        gdn = lax.GatherDimensionNumbers(
            offset_dims=(), collapsed_slice_dims=(0,), start_index_map=(0,))

        def allsum(v):
            for perm in perms:
                v = v + lax.gather(
                    v, perm[:, None], dimension_numbers=gdn, slice_sizes=(1,),
                    mode=lax.GatherScatterMode.PROMISE_IN_BOUNDS)
            return v

        def compute(b, bufs):
            m0 = _tree_sum([bufs[0][l] for l in range(L)])
            m1 = _tree_sum([bufs[1][l] for l in range(L)])
            m2 = _tree_sum([bufs[2][l] for l in range(L)])
            p = (r01[b] * r10[b] + r02[b] * r20[b] + r12[b] * r21[b]
                 + (r03[b] * m0 + r13[b] * m1 + r23[b] * m2) * INV_L)
            return allsum(p)

        issue(0, ring0, sem_a)

        def body(t, acc):
            b0 = 2 * t
            lane0 = b0 % D
            issue(b0 + 1, ring1, sem_b)
            wait3(ring0, sem_a)
            s0 = compute(b0, ring0)
            issue(jnp.minimum(b0 + 2, BPT - 1), ring0, sem_a)
            wait3(ring1, sem_b)
            s1 = compute(b0 + 1, ring1)
            acc = jnp.where(lanes == lane0, s0, acc)
            acc = jnp.where(lanes == lane0 + 1, s1, acc)

            @pl.when(t % (D // 2) == (D // 2) - 1)
            def _():
                out_v[pl.ds((t // (D // 2)) * D, D)] = acc

            return acc

        lax.fori_loop(0, BPT // 2, body, jnp.zeros((D,), jnp.float32))
        wait3(ring0, sem_a)

        pltpu.sync_copy(out_v, out.at[pl.ds(base, BPT)])

    out_flat = k(input_0.reshape(B), input_1.reshape(B), input_2.reshape(B),
                 input_3,
                 W_0_1, W_0_2, W_0_3,
                 W_1_0, W_1_2, W_1_3,
                 W_2_0, W_2_1, W_2_3,
                 W_3_0, W_3_1, W_3_2)
    return out_flat.reshape(B, 1, 1)


# speedup vs baseline: 2.1640x; 1.6818x over previous
import functools

import jax
import jax.numpy as jnp
from jax import lax
from jax.experimental import pallas as pl
from jax.experimental.pallas import tpu as pltpu
from jax.experimental.pallas import tpu_sc as plsc

B = 4096
V = 100000
D = 16
L = 50
NC = 2
NS = 16
NW = NC * NS
BPT = B // NW
INV_L = 1.0 / L


def _tree_sum(vals):
    while len(vals) > 1:
        nxt = [vals[i] + vals[i + 1] for i in range(0, len(vals) - 1, 2)]
        if len(vals) % 2:
            nxt.append(vals[-1])
        vals = nxt
    return vals[0]


def kernel(input_0, input_1, input_2, input_3,
           W_0_1, W_0_2, W_0_3,
           W_1_0, W_1_2, W_1_3,
           W_2_0, W_2_1, W_2_3,
           W_3_0, W_3_1, W_3_2):
    mesh = plsc.VectorSubcoreMesh(core_axis_name="c", subcore_axis_name="s")

    @functools.partial(
        pl.kernel,
        mesh=mesh,
        compiler_params=pltpu.CompilerParams(
            needs_layout_passes=False, use_tc_tiling_on_sc=False),
        out_type=jax.ShapeDtypeStruct((B,), jnp.float32),
        scratch_types=[
            pltpu.VMEM((BPT,), jnp.int32),
            pltpu.VMEM((BPT,), jnp.int32),
            pltpu.VMEM((BPT,), jnp.int32),
            pltpu.VMEM((BPT, L), jnp.int32),
        ] + [pltpu.VMEM((D, BPT), jnp.int32) for _ in range(3)]
          + [pltpu.VMEM((D, 136), jnp.float32) for _ in range(9)]
          + [pltpu.VMEM((L, D), jnp.float32) for _ in range(6)]
          + [
            pltpu.VMEM((BPT,), jnp.float32),
            pltpu.SemaphoreType.DMA,
            pltpu.SemaphoreType.DMA,
            pltpu.SemaphoreType.DMA,
        ],
    )
    def k(i0, i1, i2, i3,
          w01, w02, w03, w10, w12, w13, w20, w21, w23, w30, w31, w32,
          out,
          idx0_v, idx1_v, idx2_v, idx3_v,
          im0, im1, im2,
          r01, r02, r03, r10, r12, r13, r20, r21, r23,
          s0a, s1a, s2a, s0b, s1b, s2b,
          out_v,
          sem_sc, sem_a, sem_b):
        wid = lax.axis_index("s") * NC + lax.axis_index("c")
        base = wid * BPT

        pltpu.sync_copy(i0.at[pl.ds(base, BPT)], idx0_v)
        pltpu.sync_copy(i1.at[pl.ds(base, BPT)], idx1_v)
        pltpu.sync_copy(i2.at[pl.ds(base, BPT)], idx2_v)
        pltpu.sync_copy(i3.at[pl.ds(base, BPT)], idx3_v)

        for im, idxv in ((im0, idx0_v), (im1, idx1_v), (im2, idx2_v)):
            for f in range(D):
                for j in range(BPT // D):
                    im[f, pl.ds(j * D, D)] = (
                        idxv[pl.ds(j * D, D)] + f * V)

        field_tabs = ((im0, (w01, r01), (w02, r02), (w03, r03)),
                      (im1, (w10, r10), (w12, r12), (w13, r13)),
                      (im2, (w20, r20), (w21, r21), (w23, r23)))
        sc_copies = []
        for im, *tabs in field_tabs:
            for w, rbuf in tabs:
                for f in range(D):
                    sc_copies.append(pltpu.make_async_copy(
                        w.at[im.at[f]], rbuf.at[f, pl.ds(0, BPT)], sem_sc))
        for c in sc_copies:
            c.start()
        for c in sc_copies:
            c.wait()

        ring0 = (s0a, s1a, s2a)
        ring1 = (s0b, s1b, s2b)

        def issue(b, bufs, sem):
            idxrow = idx3_v.at[b]
            pltpu.make_async_copy(w30.at[idxrow], bufs[0], sem).start()
            pltpu.make_async_copy(w31.at[idxrow], bufs[1], sem).start()
            pltpu.make_async_copy(w32.at[idxrow], bufs[2], sem).start()

        def wait3(bufs, sem):
            for buf in bufs:
                pltpu.make_async_copy(w30.at[idx3_v.at[0]], buf, sem).wait()

        lanes = lax.iota(jnp.int32, D)
        perms = [lanes ^ sh for sh in (8, 4, 2, 1)]

        def colload(rbuf, b):
            bvec = jnp.zeros((D,), jnp.int32) + b
            return plsc.load_gather(rbuf, [lanes, bvec])

        gdn = lax.GatherDimensionNumbers(
            offset_dims=(), collapsed_slice_dims=(0,), start_index_map=(0,))

        def allsum(v):
            for perm in perms:
                v = v + lax.gather(
                    v, perm[:, None], dimension_numbers=gdn, slice_sizes=(1,),
                    mode=lax.GatherScatterMode.PROMISE_IN_BOUNDS)
            return v

        def compute(b, bufs):
            m0 = _tree_sum([bufs[0][l] for l in range(L)])
            m1 = _tree_sum([bufs[1][l] for l in range(L)])
            m2 = _tree_sum([bufs[2][l] for l in range(L)])
            p = (colload(r01, b) * colload(r10, b)
                 + colload(r02, b) * colload(r20, b)
                 + colload(r12, b) * colload(r21, b)
                 + (colload(r03, b) * m0 + colload(r13, b) * m1
                    + colload(r23, b) * m2) * INV_L)
            return allsum(p)

        issue(0, ring0, sem_a)

        def body(t, acc):
            b0 = 2 * t
            lane0 = b0 % D
            issue(b0 + 1, ring1, sem_b)
            wait3(ring0, sem_a)
            s0 = compute(b0, ring0)
            issue(jnp.minimum(b0 + 2, BPT - 1), ring0, sem_a)
            wait3(ring1, sem_b)
            s1 = compute(b0 + 1, ring1)
            acc = jnp.where(lanes == lane0, s0, acc)
            acc = jnp.where(lanes == lane0 + 1, s1, acc)

            @pl.when(t % (D // 2) == (D // 2) - 1)
            def _():
                out_v[pl.ds((t // (D // 2)) * D, D)] = acc

            return acc

        lax.fori_loop(0, BPT // 2, body, jnp.zeros((D,), jnp.float32))
        wait3(ring0, sem_a)

        pltpu.sync_copy(out_v, out.at[pl.ds(base, BPT)])

    out_flat = k(input_0.reshape(B), input_1.reshape(B), input_2.reshape(B),
                 input_3,
                 *[w.T.reshape(V * D) for w in (W_0_1, W_0_2, W_0_3,
                                                W_1_0, W_1_2, W_1_3,
                                                W_2_0, W_2_1, W_2_3)],
                 W_3_0, W_3_1, W_3_2)
    return out_flat.reshape(B, 1, 1)
